# speedup vs baseline: 1.6304x; 1.6304x over previous
import functools

import jax
import jax.numpy as jnp
from jax.experimental import pallas as pl

N_PAD = 10240


def _mm_attn_body(x_ref, w_ref, asrc_ref, adst_ref, h_ref, as_ref, ad_ref):
    h = x_ref[...] @ w_ref[...]
    h_ref[...] = h
    as_ref[...] = jnp.sum(h * asrc_ref[...], axis=-1)
    ad_ref[...] = jnp.sum(h * adst_ref[...], axis=-1)


def _mm_attn(x_pad, W, att_src, att_dst):
    n, d_in = x_pad.shape
    d_out = W.shape[1]
    blk = 1024
    grid = n // blk
    return pl.pallas_call(
        _mm_attn_body,
        grid=(grid,),
        in_specs=[
            pl.BlockSpec((blk, d_in), lambda i: (i, 0)),
            pl.BlockSpec((d_in, d_out), lambda i: (0, 0)),
            pl.BlockSpec((1, d_out), lambda i: (0, 0)),
            pl.BlockSpec((1, d_out), lambda i: (0, 0)),
        ],
        out_specs=[
            pl.BlockSpec((blk, d_out), lambda i: (i, 0)),
            pl.BlockSpec((blk,), lambda i: (i,)),
            pl.BlockSpec((blk,), lambda i: (i,)),
        ],
        out_shape=[
            jax.ShapeDtypeStruct((n, d_out), jnp.float32),
            jax.ShapeDtypeStruct((n,), jnp.float32),
            jax.ShapeDtypeStruct((n,), jnp.float32),
        ],
    )(x_pad, W, att_src.reshape(1, -1), att_dst.reshape(1, -1))


def _edge_aggregate(h, a_src, a_dst, src, dst, n):
    alpha = a_src[src] + a_dst[dst]
    alpha = jnp.where(alpha >= 0, alpha, 0.2 * alpha)
    s = jnp.exp(alpha)
    denom = jax.ops.segment_sum(s, dst, num_segments=n)
    msg = h[src] * s[:, None]
    out = jax.ops.segment_sum(msg, dst, num_segments=n)
    return out / (denom + 1e-16)[:, None]


def _final_body(h_ref, wl_ref, u_ref, v_ref):
    h = h_ref[...]
    wl = wl_ref[...]
    u_ref[...] = h @ wl[:128, :]
    v_ref[...] = h @ wl[128:, :]


def _final_uv(h_pad, Wl):
    n = h_pad.shape[0]
    wl_pad = jnp.zeros((256, 128), jnp.float32).at[:, 0].set(Wl[:, 0])
    blk = 1024
    u, v = pl.pallas_call(
        _final_body,
        grid=(n // blk,),
        in_specs=[
            pl.BlockSpec((blk, 128), lambda i: (i, 0)),
            pl.BlockSpec((256, 128), lambda i: (0, 0)),
        ],
        out_specs=[
            pl.BlockSpec((blk, 128), lambda i: (i, 0)),
            pl.BlockSpec((blk, 128), lambda i: (i, 0)),
        ],
        out_shape=[
            jax.ShapeDtypeStruct((n, 128), jnp.float32),
            jax.ShapeDtypeStruct((n, 128), jnp.float32),
        ],
    )(h_pad, wl_pad)
    return u[:, 0], v[:, 0]


def kernel(g, features, mask, W1, att_src1, att_dst1, b1,
           W2, att_src2, att_dst2, b2, Wl, bl):
    n = features.shape[0]
    loop = jnp.arange(n, dtype=g.dtype)
    src = jnp.concatenate([g[0], loop])
    dst = jnp.concatenate([g[1], loop])

    x_pad = jnp.zeros((N_PAD, features.shape[1]), jnp.float32).at[:n].set(features)

    h1, as1, ad1 = _mm_attn(x_pad, W1, att_src1, att_dst1)
    out1 = _edge_aggregate(h1[:n], as1[:n], ad1[:n], src, dst, n)
    h1r = jax.nn.relu(out1 + b1)

    h1r_pad = jnp.zeros((N_PAD, h1r.shape[1]), jnp.float32).at[:n].set(h1r)
    h2, as2, ad2 = _mm_attn(h1r_pad, W2, att_src2, att_dst2)
    out2 = _edge_aggregate(h2[:n], as2[:n], ad2[:n], src, dst, n) + b2

    out2_pad = jnp.zeros((N_PAD, out2.shape[1]), jnp.float32).at[:n].set(out2)
    u, v = _final_uv(out2_pad, Wl)
    m = mask.T
    logits = u[m[0]] + v[m[1]] + bl[0]
    return jax.nn.sigmoid(logits)[:, None]


# speedup vs baseline: 10.2920x; 6.3124x over previous
import functools

import jax
import jax.numpy as jnp
from jax import lax
from jax.experimental import pallas as pl
from jax.experimental.pallas import tpu as pltpu
from jax.experimental.pallas import tpu_sc as plsc

N_PAD = 10240
NC = 2
NS = 16
NW = NC * NS


def _pair_score_body(m0_hbm, m1_hbm, u_hbm, v_hbm, out_hbm,
                     m0_v, m1_v, u_v, v_v, o_v):
    wid = lax.axis_index("s") * NC + lax.axis_index("c")
    chunk = m0_v.shape[0]
    base = wid * chunk
    pltpu.sync_copy(u_hbm, u_v)
    pltpu.sync_copy(v_hbm, v_v)
    pltpu.sync_copy(m0_hbm.at[pl.ds(base, chunk)], m0_v)
    pltpu.sync_copy(m1_hbm.at[pl.ds(base, chunk)], m1_v)

    def vec(j, _):
        i0 = m0_v[pl.ds(j * 16, 16)]
        i1 = m1_v[pl.ds(j * 16, 16)]
        x = plsc.load_gather(u_v, [i0]) + plsc.load_gather(v_v, [i1])
        o_v[pl.ds(j * 16, 16)] = 1.0 / (1.0 + jnp.exp(-x))
        return _

    lax.fori_loop(0, chunk // 16, vec, 0)
    pltpu.sync_copy(o_v, out_hbm.at[pl.ds(base, chunk)])


def _pair_score(m0, m1, u, v):
    mp = m0.shape[0]
    n = u.shape[0]
    chunk = mp // NW
    mesh = plsc.VectorSubcoreMesh(core_axis_name="c", subcore_axis_name="s")
    f = pl.kernel(
        _pair_score_body,
        out_type=jax.ShapeDtypeStruct((mp,), jnp.float32),
        mesh=mesh,
        compiler_params=pltpu.CompilerParams(
            use_tc_tiling_on_sc=False, needs_layout_passes=False),
        scratch_types=[
            pltpu.VMEM((chunk,), jnp.int32),
            pltpu.VMEM((chunk,), jnp.int32),
            pltpu.VMEM((n,), jnp.float32),
            pltpu.VMEM((n,), jnp.float32),
            pltpu.VMEM((chunk,), jnp.float32),
        ],
    )
    return f(m0, m1, u, v)


HALF = 5000
ACC_R = 5008
DUMP = 5000
K = 64
BS = 2064
NA = 10016
SB = 14

_SC_PARAMS = dict(
    compiler_params=pltpu.CompilerParams(
        use_tc_tiling_on_sc=False, needs_layout_passes=False),
)


def _phase_a_body_factory(CHUNK, LISTSZ):
    def body(src_hbm, dst_hbm, asrc_hbm, adst_hbm, plist_hbm, slist_hbm,
             ebuf_s, ebuf_d, asrc_v, adst_v, lp, ls):
        c = lax.axis_index("c")
        sid = lax.axis_index("s")
        base = c * HALF
        pltpu.sync_copy(asrc_hbm, asrc_v)
        pltpu.sync_copy(adst_hbm, adst_v)
        chunk0 = sid * CHUNK

        def stage(b, cnt):
            eoff = chunk0 + b * BS
            pltpu.sync_copy(src_hbm.at[pl.ds(eoff, BS)], ebuf_s)
            pltpu.sync_copy(dst_hbm.at[pl.ds(eoff, BS)], ebuf_d)

            def vec(j, cnt):
                sv = ebuf_s[pl.ds(j * 16, 16)]
                dv = ebuf_d[pl.ds(j * 16, 16)]
                a = (plsc.load_gather(asrc_v, [sv])
                     + plsc.load_gather(adst_v, [dv]))
                a = jnp.where(a >= 0.0, a, a * jnp.float32(0.2))
                s = jnp.exp(a)
                own = (dv >= base) & (dv < base + HALF)
                inc = plsc.cumsum(own.astype(jnp.int32))
                pos = cnt + inc - 1
                packed = sv | ((dv - base) << SB)
                plsc.store_scatter(lp, [pos], packed, mask=own)
                plsc.store_scatter(ls, [pos], s, mask=own)
                return cnt + jnp.max(inc)

            return lax.fori_loop(0, BS // 16, vec, cnt)

        cnt = lax.fori_loop(0, CHUNK // BS, stage, jnp.int32(0))

        cntk = ((cnt + K - 1) >> 6) << 6
        iota = lax.iota(jnp.int32, 16)
        dump = jnp.full((16,), DUMP << SB, jnp.int32)
        sent = jnp.full((16,), -1, jnp.int32)
        zero = jnp.zeros((16,), jnp.float32)
        for t in range(K // 16):
            pos = cnt + t * 16 + iota
            m = pos < cntk
            plsc.store_scatter(lp, [pos], dump, mask=m)
            plsc.store_scatter(ls, [pos], zero, mask=m)
        for t in range(K // 16):
            plsc.store_scatter(lp, [cntk + t * 16 + iota], sent)

        pltpu.sync_copy(lp, plist_hbm.at[c, sid])
        pltpu.sync_copy(ls, slist_hbm.at[c, sid])

    return body


def _phase_b_body_factory(D, LISTSZ):
    W = D + 16

    def body(plist_hbm, slist_hbm, h_hbm, out_hbm,
             pch, src_ch, dst_ch, s_ch, gbuf, sbuf, acc, sem):
        c = lax.axis_index("c")
        sid = lax.axis_index("s")

        def zrow(r, t):
            for cc in range(W // 16):
                sbuf[r, pl.ds(cc * 16, 16)] = jnp.zeros((16,), jnp.float32)
            return t
        lax.fori_loop(0, K, zrow, 0)
        stripe0 = sid * (ACC_R // NS)
        n_full = (ACC_R // NS) // K
        for t in range(n_full):
            pltpu.sync_copy(sbuf, acc.at[pl.ds(stripe0 + t * K, K)])
        rem = (ACC_R // NS) - n_full * K
        if rem:
            pltpu.sync_copy(sbuf.at[pl.ds(0, rem)],
                            acc.at[pl.ds(stripe0 + n_full * K, rem)])
        plsc.subcore_barrier()

        def bchunk(st):
            i, _ = st
            off = i * K
            pltpu.sync_copy(plist_hbm.at[c, sid, pl.ds(off, K)], pch)
            v0 = pch[pl.ds(0, 16)]
            v1 = pch[pl.ds(16, 16)]
            v2 = pch[pl.ds(32, 16)]
            v3 = pch[pl.ds(48, 16)]
            go = jnp.max(jnp.maximum(jnp.maximum(v0, v1),
                                     jnp.maximum(v2, v3))) >= 0

            @pl.when(go)
            def _():
                pltpu.sync_copy(slist_hbm.at[c, sid, pl.ds(off, K)], s_ch)
                for q, vq in enumerate((v0, v1, v2, v3)):
                    src_ch[pl.ds(q * 16, 16)] = vq & ((1 << SB) - 1)
                    dst_ch[pl.ds(q * 16, 16)] = vq >> SB
                pltpu.async_copy(h_hbm.at[src_ch], gbuf, sem).wait()

                def row(r, t2):
                    sb = plsc.load_gather(s_ch, [lax.broadcast(r, (16,))])
                    for cc in range(D // 16):
                        gv = gbuf[r, pl.ds(cc * 16, 16)]
                        sbuf[r, pl.ds(cc * 16, 16)] = gv * sb
                    sbuf[r, pl.ds(D, 16)] = sb
                    return t2

                lax.fori_loop(0, K, row, 0)
                pltpu.sync_copy(sbuf, acc.at[dst_ch], add=True)

            return i + 1, go

        lax.while_loop(lambda st: st[1], bchunk, (jnp.int32(0), True))
        plsc.subcore_barrier()

        @pl.when(sid < NS - 1)
        def _():
            pltpu.sync_copy(acc.at[pl.ds(sid * 313, 313)],
                            out_hbm.at[c, pl.ds(sid * 313, 313)])

        @pl.when(sid == NS - 1)
        def _():
            pltpu.sync_copy(acc.at[pl.ds(15 * 313, HALF - 15 * 313)],
                            out_hbm.at[c, pl.ds(15 * 313, HALF - 15 * 313)])

    return body


def _edge_aggregate_sc(h, asrc, adst, src_pad, dst_pad):
    D = h.shape[1]
    W = D + 16
    E_PAD = src_pad.shape[0]
    CHUNK = E_PAD // NS
    LISTSZ = CHUNK + 2 * K
    mesh = plsc.VectorSubcoreMesh(core_axis_name="c", subcore_axis_name="s")

    fa = pl.kernel(
        _phase_a_body_factory(CHUNK, LISTSZ),
        out_type=[
            jax.ShapeDtypeStruct((NC, NS, LISTSZ), jnp.int32),
            jax.ShapeDtypeStruct((NC, NS, LISTSZ), jnp.float32),
        ],
        mesh=mesh,
        scratch_types=[
            pltpu.VMEM((BS,), jnp.int32),
            pltpu.VMEM((BS,), jnp.int32),
            pltpu.VMEM((NA,), jnp.float32),
            pltpu.VMEM((NA,), jnp.float32),
            pltpu.VMEM((LISTSZ,), jnp.int32),
            pltpu.VMEM((LISTSZ,), jnp.float32),
        ],
        **_SC_PARAMS,
    )
    plist, slist = fa(src_pad, dst_pad, asrc, adst)

    fb = pl.kernel(
        _phase_b_body_factory(D, LISTSZ),
        out_type=jax.ShapeDtypeStruct((NC, HALF, W), jnp.float32),
        mesh=mesh,
        scratch_types=[
            pltpu.VMEM((K,), jnp.int32),
            pltpu.VMEM((K,), jnp.int32),
            pltpu.VMEM((K,), jnp.int32),
            pltpu.VMEM((K,), jnp.float32),
            pltpu.VMEM((K, D), jnp.float32),
            pltpu.VMEM((K, W), jnp.float32),
            pltpu.VMEM_SHARED((ACC_R, W), jnp.float32),
            pltpu.SemaphoreType.DMA,
        ],
        **_SC_PARAMS,
    )
    acc = fb(plist, slist, h)
    return acc.reshape(NC * HALF, W)


def _mm_attn_body(x_ref, w_ref, asrc_ref, adst_ref, h_ref, as_ref, ad_ref):
    h = x_ref[...] @ w_ref[...]
    h_ref[...] = h
    as_ref[...] = jnp.sum(h * asrc_ref[...], axis=-1)
    ad_ref[...] = jnp.sum(h * adst_ref[...], axis=-1)


def _mm_attn(x_pad, W, att_src, att_dst):
    n, d_in = x_pad.shape
    d_out = W.shape[1]
    blk = 1024
    grid = n // blk
    return pl.pallas_call(
        _mm_attn_body,
        grid=(grid,),
        in_specs=[
            pl.BlockSpec((blk, d_in), lambda i: (i, 0)),
            pl.BlockSpec((d_in, d_out), lambda i: (0, 0)),
            pl.BlockSpec((1, d_out), lambda i: (0, 0)),
            pl.BlockSpec((1, d_out), lambda i: (0, 0)),
        ],
        out_specs=[
            pl.BlockSpec((blk, d_out), lambda i: (i, 0)),
            pl.BlockSpec((blk,), lambda i: (i,)),
            pl.BlockSpec((blk,), lambda i: (i,)),
        ],
        out_shape=[
            jax.ShapeDtypeStruct((n, d_out), jnp.float32),
            jax.ShapeDtypeStruct((n,), jnp.float32),
            jax.ShapeDtypeStruct((n,), jnp.float32),
        ],
    )(x_pad, W, att_src.reshape(1, -1), att_dst.reshape(1, -1))


def _norm_mm_attn_body(acc_ref, b_ref, w_ref, asrc_ref, adst_ref,
                       h_ref, as_ref, ad_ref, *, d_prev):
    x = acc_ref[...]
    num = x[:, :d_prev]
    den = x[:, d_prev:d_prev + 1]
    h_in = jax.nn.relu(num / (den + 1e-16) + b_ref[...])
    h = h_in @ w_ref[...]
    h_ref[...] = h
    as_ref[...] = jnp.sum(h * asrc_ref[...], axis=-1)
    ad_ref[...] = jnp.sum(h * adst_ref[...], axis=-1)


def _norm_mm_attn(acc_pad, b, W, att_src, att_dst):
    n, w_in = acc_pad.shape
    d_prev = b.shape[0]
    d_out = W.shape[1]
    blk = 1024
    return pl.pallas_call(
        functools.partial(_norm_mm_attn_body, d_prev=d_prev),
        grid=(n // blk,),
        in_specs=[
            pl.BlockSpec((blk, w_in), lambda i: (i, 0)),
            pl.BlockSpec((1, d_prev), lambda i: (0, 0)),
            pl.BlockSpec((d_prev, d_out), lambda i: (0, 0)),
            pl.BlockSpec((1, d_out), lambda i: (0, 0)),
            pl.BlockSpec((1, d_out), lambda i: (0, 0)),
        ],
        out_specs=[
            pl.BlockSpec((blk, d_out), lambda i: (i, 0)),
            pl.BlockSpec((blk,), lambda i: (i,)),
            pl.BlockSpec((blk,), lambda i: (i,)),
        ],
        out_shape=[
            jax.ShapeDtypeStruct((n, d_out), jnp.float32),
            jax.ShapeDtypeStruct((n,), jnp.float32),
            jax.ShapeDtypeStruct((n,), jnp.float32),
        ],
    )(acc_pad, b.reshape(1, -1), W,
      att_src.reshape(1, -1), att_dst.reshape(1, -1))


def _norm_uv_body(acc_ref, b_ref, wlu_ref, wlv_ref, u_ref, v_ref, *, d_prev):
    x = acc_ref[...]
    num = x[:, :d_prev]
    den = x[:, d_prev:d_prev + 1]
    o = num / (den + 1e-16) + b_ref[...]
    u_ref[...] = jnp.sum(o * wlu_ref[...], axis=-1)
    v_ref[...] = jnp.sum(o * wlv_ref[...], axis=-1)


def _norm_uv(acc_pad, b, Wl):
    n, w_in = acc_pad.shape
    d_prev = b.shape[0]
    blk = 1024
    return pl.pallas_call(
        functools.partial(_norm_uv_body, d_prev=d_prev),
        grid=(n // blk,),
        in_specs=[
            pl.BlockSpec((blk, w_in), lambda i: (i, 0)),
            pl.BlockSpec((1, d_prev), lambda i: (0, 0)),
            pl.BlockSpec((1, d_prev), lambda i: (0, 0)),
            pl.BlockSpec((1, d_prev), lambda i: (0, 0)),
        ],
        out_specs=[
            pl.BlockSpec((blk,), lambda i: (i,)),
            pl.BlockSpec((blk,), lambda i: (i,)),
        ],
        out_shape=[
            jax.ShapeDtypeStruct((n,), jnp.float32),
            jax.ShapeDtypeStruct((n,), jnp.float32),
        ],
    )(acc_pad, b.reshape(1, -1),
      Wl[:d_prev, 0].reshape(1, -1), Wl[d_prev:, 0].reshape(1, -1))


def kernel(g, features, mask, W1, att_src1, att_dst1, b1,
           W2, att_src2, att_dst2, b2, Wl, bl):
    n = features.shape[0]
    e = g.shape[1]
    e_pad = ((e + n + NS * BS - 1) // (NS * BS)) * (NS * BS)
    loop = jnp.arange(n, dtype=g.dtype)
    src_pad = jnp.concatenate(
        [g[0], loop, jnp.zeros((e_pad - e - n,), g.dtype)])
    dst_pad = jnp.concatenate(
        [g[1], loop, jnp.full((e_pad - e - n,), n, g.dtype)])

    x_pad = jnp.zeros((N_PAD, features.shape[1]), jnp.float32).at[:n].set(features)

    h1, as1, ad1 = _mm_attn(x_pad, W1, att_src1, att_dst1)
    acc1 = _edge_aggregate_sc(h1, as1[:NA], ad1[:NA], src_pad, dst_pad)
    acc1_pad = jnp.zeros((N_PAD, acc1.shape[1]), jnp.float32).at[:n].set(acc1)

    h2, as2, ad2 = _norm_mm_attn(acc1_pad, b1, W2, att_src2, att_dst2)
    acc2 = _edge_aggregate_sc(h2, as2[:NA], ad2[:NA], src_pad, dst_pad)
    acc2_pad = jnp.zeros((N_PAD, acc2.shape[1]), jnp.float32).at[:n].set(acc2)

    u, v = _norm_uv(acc2_pad, b2, Wl)
    u10 = u + bl[0]
    v10 = v
    nm = mask.shape[0]
    m_pad = ((nm + NW * 16 - 1) // (NW * 16)) * (NW * 16)
    m0 = jnp.pad(mask[:, 0], (0, m_pad - nm))
    m1 = jnp.pad(mask[:, 1], (0, m_pad - nm))
    p = _pair_score(m0, m1, u10, v10)
    return p[:nm, None]


# speedup vs baseline: 11.7333x; 1.1400x over previous
import functools

import jax
import jax.numpy as jnp
from jax import lax
from jax.experimental import pallas as pl
from jax.experimental.pallas import tpu as pltpu
from jax.experimental.pallas import tpu_sc as plsc

N_PAD = 10240
NC = 2
NS = 16
NW = NC * NS


def _pair_score_body(m0_hbm, m1_hbm, u_hbm, v_hbm, out_hbm,
                     m0_v, m1_v, u_v, v_v, o_v):
    wid = lax.axis_index("s") * NC + lax.axis_index("c")
    chunk = m0_v.shape[0]
    base = wid * chunk
    pltpu.sync_copy(u_hbm, u_v)
    pltpu.sync_copy(v_hbm, v_v)
    pltpu.sync_copy(m0_hbm.at[pl.ds(base, chunk)], m0_v)
    pltpu.sync_copy(m1_hbm.at[pl.ds(base, chunk)], m1_v)

    def vec(j, _):
        i0 = m0_v[pl.ds(j * 16, 16)]
        i1 = m1_v[pl.ds(j * 16, 16)]
        x = plsc.load_gather(u_v, [i0]) + plsc.load_gather(v_v, [i1])
        o_v[pl.ds(j * 16, 16)] = 1.0 / (1.0 + jnp.exp(-x))
        return _

    lax.fori_loop(0, chunk // 16, vec, 0)
    pltpu.sync_copy(o_v, out_hbm.at[pl.ds(base, chunk)])


def _pair_score(m0, m1, u, v):
    mp = m0.shape[0]
    n = u.shape[0]
    chunk = mp // NW
    mesh = plsc.VectorSubcoreMesh(core_axis_name="c", subcore_axis_name="s")
    f = pl.kernel(
        _pair_score_body,
        out_type=jax.ShapeDtypeStruct((mp,), jnp.float32),
        mesh=mesh,
        compiler_params=pltpu.CompilerParams(
            use_tc_tiling_on_sc=False, needs_layout_passes=False),
        scratch_types=[
            pltpu.VMEM((chunk,), jnp.int32),
            pltpu.VMEM((chunk,), jnp.int32),
            pltpu.VMEM((n,), jnp.float32),
            pltpu.VMEM((n,), jnp.float32),
            pltpu.VMEM((chunk,), jnp.float32),
        ],
    )
    return f(m0, m1, u, v)


HALF = 5000
ACC_R = 5008
DUMP = 5000
K = 48
BS = 2064
NA = 10016
SB = 14

_SC_PARAMS = dict(
    compiler_params=pltpu.CompilerParams(
        use_tc_tiling_on_sc=False, needs_layout_passes=False),
)


def _phase_a_body_factory(CHUNK, LISTSZ):
    def body(src_hbm, dst_hbm, asrc_hbm, adst_hbm, plist_hbm, slist_hbm,
             ebuf_s, ebuf_d, asrc_v, adst_v, lp, ls):
        c = lax.axis_index("c")
        sid = lax.axis_index("s")
        base = c * HALF
        pltpu.sync_copy(asrc_hbm, asrc_v)
        pltpu.sync_copy(adst_hbm, adst_v)
        chunk0 = sid * CHUNK

        def stage(b, cnt):
            eoff = chunk0 + b * BS
            pltpu.sync_copy(src_hbm.at[pl.ds(eoff, BS)], ebuf_s)
            pltpu.sync_copy(dst_hbm.at[pl.ds(eoff, BS)], ebuf_d)

            def vec(j, cnt):
                sv = ebuf_s[pl.ds(j * 16, 16)]
                dv = ebuf_d[pl.ds(j * 16, 16)]
                a = (plsc.load_gather(asrc_v, [sv])
                     + plsc.load_gather(adst_v, [dv]))
                a = jnp.where(a >= 0.0, a, a * jnp.float32(0.2))
                s = jnp.exp(a)
                own = (dv >= base) & (dv < base + HALF)
                inc = plsc.cumsum(own.astype(jnp.int32))
                pos = cnt + inc - 1
                packed = sv | ((dv - base) << SB)
                plsc.store_scatter(lp, [pos], packed, mask=own)
                plsc.store_scatter(ls, [pos], s, mask=own)
                return cnt + jnp.max(inc)

            return lax.fori_loop(0, BS // 16, vec, cnt)

        cnt = lax.fori_loop(0, CHUNK // BS, stage, jnp.int32(0))

        cntk = ((cnt + K - 1) // K) * K
        iota = lax.iota(jnp.int32, 16)
        dump = jnp.full((16,), DUMP << SB, jnp.int32)
        sent = jnp.full((16,), -1, jnp.int32)
        zero = jnp.zeros((16,), jnp.float32)
        for t in range(K // 16):
            pos = cnt + t * 16 + iota
            m = pos < cntk
            plsc.store_scatter(lp, [pos], dump, mask=m)
            plsc.store_scatter(ls, [pos], zero, mask=m)
        for t in range(K // 16):
            plsc.store_scatter(lp, [cntk + t * 16 + iota], sent)

        pltpu.sync_copy(lp, plist_hbm.at[c, sid])
        pltpu.sync_copy(ls, slist_hbm.at[c, sid])

    return body


def _phase_b_body_factory(D, LISTSZ):
    W = D + 16

    def body(plist_hbm, slist_hbm, h_hbm, out_hbm,
             pch0, src0, dst0, s0, gbuf0, sem0,
             pch1, src1, dst1, s1, gbuf1, sem1,
             sbuf, acc):
        c = lax.axis_index("c")
        sid = lax.axis_index("s")

        def zrow(r, t):
            for cc in range(W // 16):
                sbuf[r, pl.ds(cc * 16, 16)] = jnp.zeros((16,), jnp.float32)
            return t
        lax.fori_loop(0, K, zrow, 0)
        stripe0 = sid * (ACC_R // NS)
        n_full = (ACC_R // NS) // K
        for t in range(n_full):
            pltpu.sync_copy(sbuf, acc.at[pl.ds(stripe0 + t * K, K)])
        rem = (ACC_R // NS) - n_full * K
        if rem:
            pltpu.sync_copy(sbuf.at[pl.ds(0, rem)],
                            acc.at[pl.ds(stripe0 + n_full * K, rem)])
        plsc.subcore_barrier()

        def peek(pch, i):
            pltpu.sync_copy(plist_hbm.at[c, sid, pl.ds(i * K, K)], pch)
            vs = [pch[pl.ds(q * 16, 16)] for q in range(K // 16)]
            mx = vs[0]
            for v in vs[1:]:
                mx = jnp.maximum(mx, v)
            return jnp.max(mx) >= 0, vs

        def prep(vs, src_ch, dst_ch, s_ch, gbuf, sem, i):
            pltpu.sync_copy(slist_hbm.at[c, sid, pl.ds(i * K, K)], s_ch)
            for q, vq in enumerate(vs):
                src_ch[pl.ds(q * 16, 16)] = vq & ((1 << SB) - 1)
                dst_ch[pl.ds(q * 16, 16)] = vq >> SB
            pltpu.async_copy(h_hbm.at[src_ch], gbuf, sem)

        def process(src_ch, dst_ch, s_ch, gbuf, sem):
            pltpu.make_async_copy(h_hbm.at[src_ch], gbuf, sem).wait()

            def row(r, t2):
                sb = plsc.load_gather(s_ch, [lax.broadcast(r, (16,))])
                for cc in range(D // 16):
                    sbuf[r, pl.ds(cc * 16, 16)] = gbuf[r, pl.ds(cc * 16, 16)] * sb
                sbuf[r, pl.ds(D, 16)] = sb
                return t2

            lax.fori_loop(0, K, row, 0)
            pltpu.sync_copy(sbuf, acc.at[dst_ch], add=True)

        gA, vs0 = peek(pch0, jnp.int32(0))

        @pl.when(gA)
        def _():
            prep(vs0, src0, dst0, s0, gbuf0, sem0, jnp.int32(0))
        gB_raw, vs1 = peek(pch1, jnp.int32(1))
        gB = gA & gB_raw

        @pl.when(gB)
        def _():
            prep(vs1, src1, dst1, s1, gbuf1, sem1, jnp.int32(1))

        def pair(st):
            i, ga, gb = st
            gc_raw, vsn0 = peek(pch0, i + 2)
            gc = gb & gc_raw
            process(src0, dst0, s0, gbuf0, sem0)

            @pl.when(gc)
            def _():
                prep(vsn0, src0, dst0, s0, gbuf0, sem0, i + 2)

            @pl.when(gb)
            def _():
                process(src1, dst1, s1, gbuf1, sem1)
            gd_raw, vsn1 = peek(pch1, i + 3)
            gd = gc & gd_raw

            @pl.when(gd)
            def _():
                prep(vsn1, src1, dst1, s1, gbuf1, sem1, i + 3)

            return i + 2, gc, gd

        lax.while_loop(lambda st: st[1], pair, (jnp.int32(0), gA, gB))
        plsc.subcore_barrier()

        @pl.when(sid < NS - 1)
        def _():
            pltpu.sync_copy(acc.at[pl.ds(sid * 313, 313)],
                            out_hbm.at[c, pl.ds(sid * 313, 313)])

        @pl.when(sid == NS - 1)
        def _():
            pltpu.sync_copy(acc.at[pl.ds(15 * 313, HALF - 15 * 313)],
                            out_hbm.at[c, pl.ds(15 * 313, HALF - 15 * 313)])

    return body


def _edge_aggregate_sc(h, asrc, adst, src_pad, dst_pad):
    D = h.shape[1]
    W = D + 16
    E_PAD = src_pad.shape[0]
    CHUNK = E_PAD // NS
    LISTSZ = CHUNK + 3 * K
    mesh = plsc.VectorSubcoreMesh(core_axis_name="c", subcore_axis_name="s")

    fa = pl.kernel(
        _phase_a_body_factory(CHUNK, LISTSZ),
        out_type=[
            jax.ShapeDtypeStruct((NC, NS, LISTSZ), jnp.int32),
            jax.ShapeDtypeStruct((NC, NS, LISTSZ), jnp.float32),
        ],
        mesh=mesh,
        scratch_types=[
            pltpu.VMEM((BS,), jnp.int32),
            pltpu.VMEM((BS,), jnp.int32),
            pltpu.VMEM((NA,), jnp.float32),
            pltpu.VMEM((NA,), jnp.float32),
            pltpu.VMEM((LISTSZ,), jnp.int32),
            pltpu.VMEM((LISTSZ,), jnp.float32),
        ],
        **_SC_PARAMS,
    )
    plist, slist = fa(src_pad, dst_pad, asrc, adst)

    fb = pl.kernel(
        _phase_b_body_factory(D, LISTSZ),
        out_type=jax.ShapeDtypeStruct((NC, HALF, W), jnp.float32),
        mesh=mesh,
        scratch_types=[
            pltpu.VMEM((K,), jnp.int32),
            pltpu.VMEM((K,), jnp.int32),
            pltpu.VMEM((K,), jnp.int32),
            pltpu.VMEM((K,), jnp.float32),
            pltpu.VMEM((K, D), jnp.float32),
            pltpu.SemaphoreType.DMA,
            pltpu.VMEM((K,), jnp.int32),
            pltpu.VMEM((K,), jnp.int32),
            pltpu.VMEM((K,), jnp.int32),
            pltpu.VMEM((K,), jnp.float32),
            pltpu.VMEM((K, D), jnp.float32),
            pltpu.SemaphoreType.DMA,
            pltpu.VMEM((K, W), jnp.float32),
            pltpu.VMEM_SHARED((ACC_R, W), jnp.float32),
        ],
        **_SC_PARAMS,
    )
    acc = fb(plist, slist, h)
    return acc.reshape(NC * HALF, W)


def _mm_attn_body(x_ref, w_ref, asrc_ref, adst_ref, h_ref, as_ref, ad_ref):
    h = x_ref[...] @ w_ref[...]
    h_ref[...] = h
    as_ref[...] = jnp.sum(h * asrc_ref[...], axis=-1)
    ad_ref[...] = jnp.sum(h * adst_ref[...], axis=-1)


def _mm_attn(x_pad, W, att_src, att_dst):
    n, d_in = x_pad.shape
    d_out = W.shape[1]
    blk = 1024
    grid = n // blk
    return pl.pallas_call(
        _mm_attn_body,
        grid=(grid,),
        in_specs=[
            pl.BlockSpec((blk, d_in), lambda i: (i, 0)),
            pl.BlockSpec((d_in, d_out), lambda i: (0, 0)),
            pl.BlockSpec((1, d_out), lambda i: (0, 0)),
            pl.BlockSpec((1, d_out), lambda i: (0, 0)),
        ],
        out_specs=[
            pl.BlockSpec((blk, d_out), lambda i: (i, 0)),
            pl.BlockSpec((blk,), lambda i: (i,)),
            pl.BlockSpec((blk,), lambda i: (i,)),
        ],
        out_shape=[
            jax.ShapeDtypeStruct((n, d_out), jnp.float32),
            jax.ShapeDtypeStruct((n,), jnp.float32),
            jax.ShapeDtypeStruct((n,), jnp.float32),
        ],
    )(x_pad, W, att_src.reshape(1, -1), att_dst.reshape(1, -1))


def _norm_mm_attn_body(acc_ref, b_ref, w_ref, asrc_ref, adst_ref,
                       h_ref, as_ref, ad_ref, *, d_prev):
    x = acc_ref[...]
    num = x[:, :d_prev]
    den = x[:, d_prev:d_prev + 1]
    h_in = jax.nn.relu(num / (den + 1e-16) + b_ref[...])
    h = h_in @ w_ref[...]
    h_ref[...] = h
    as_ref[...] = jnp.sum(h * asrc_ref[...], axis=-1)
    ad_ref[...] = jnp.sum(h * adst_ref[...], axis=-1)


def _norm_mm_attn(acc_pad, b, W, att_src, att_dst):
    n, w_in = acc_pad.shape
    d_prev = b.shape[0]
    d_out = W.shape[1]
    blk = 1024
    return pl.pallas_call(
        functools.partial(_norm_mm_attn_body, d_prev=d_prev),
        grid=(n // blk,),
        in_specs=[
            pl.BlockSpec((blk, w_in), lambda i: (i, 0)),
            pl.BlockSpec((1, d_prev), lambda i: (0, 0)),
            pl.BlockSpec((d_prev, d_out), lambda i: (0, 0)),
            pl.BlockSpec((1, d_out), lambda i: (0, 0)),
            pl.BlockSpec((1, d_out), lambda i: (0, 0)),
        ],
        out_specs=[
            pl.BlockSpec((blk, d_out), lambda i: (i, 0)),
            pl.BlockSpec((blk,), lambda i: (i,)),
            pl.BlockSpec((blk,), lambda i: (i,)),
        ],
        out_shape=[
            jax.ShapeDtypeStruct((n, d_out), jnp.float32),
            jax.ShapeDtypeStruct((n,), jnp.float32),
            jax.ShapeDtypeStruct((n,), jnp.float32),
        ],
    )(acc_pad, b.reshape(1, -1), W,
      att_src.reshape(1, -1), att_dst.reshape(1, -1))


def _norm_uv_body(acc_ref, b_ref, wlu_ref, wlv_ref, u_ref, v_ref, *, d_prev):
    x = acc_ref[...]
    num = x[:, :d_prev]
    den = x[:, d_prev:d_prev + 1]
    o = num / (den + 1e-16) + b_ref[...]
    u_ref[...] = jnp.sum(o * wlu_ref[...], axis=-1)
    v_ref[...] = jnp.sum(o * wlv_ref[...], axis=-1)


def _norm_uv(acc_pad, b, Wl):
    n, w_in = acc_pad.shape
    d_prev = b.shape[0]
    blk = 1024
    return pl.pallas_call(
        functools.partial(_norm_uv_body, d_prev=d_prev),
        grid=(n // blk,),
        in_specs=[
            pl.BlockSpec((blk, w_in), lambda i: (i, 0)),
            pl.BlockSpec((1, d_prev), lambda i: (0, 0)),
            pl.BlockSpec((1, d_prev), lambda i: (0, 0)),
            pl.BlockSpec((1, d_prev), lambda i: (0, 0)),
        ],
        out_specs=[
            pl.BlockSpec((blk,), lambda i: (i,)),
            pl.BlockSpec((blk,), lambda i: (i,)),
        ],
        out_shape=[
            jax.ShapeDtypeStruct((n,), jnp.float32),
            jax.ShapeDtypeStruct((n,), jnp.float32),
        ],
    )(acc_pad, b.reshape(1, -1),
      Wl[:d_prev, 0].reshape(1, -1), Wl[d_prev:, 0].reshape(1, -1))


def kernel(g, features, mask, W1, att_src1, att_dst1, b1,
           W2, att_src2, att_dst2, b2, Wl, bl):
    n = features.shape[0]
    e = g.shape[1]
    e_pad = ((e + n + NS * BS - 1) // (NS * BS)) * (NS * BS)
    loop = jnp.arange(n, dtype=g.dtype)
    src_pad = jnp.concatenate(
        [g[0], loop, jnp.zeros((e_pad - e - n,), g.dtype)])
    dst_pad = jnp.concatenate(
        [g[1], loop, jnp.full((e_pad - e - n,), n, g.dtype)])

    x_pad = jnp.zeros((N_PAD, features.shape[1]), jnp.float32).at[:n].set(features)

    h1, as1, ad1 = _mm_attn(x_pad, W1, att_src1, att_dst1)
    acc1 = _edge_aggregate_sc(h1, as1[:NA], ad1[:NA], src_pad, dst_pad)
    acc1_pad = jnp.zeros((N_PAD, acc1.shape[1]), jnp.float32).at[:n].set(acc1)

    h2, as2, ad2 = _norm_mm_attn(acc1_pad, b1, W2, att_src2, att_dst2)
    acc2 = _edge_aggregate_sc(h2, as2[:NA], ad2[:NA], src_pad, dst_pad)
    acc2_pad = jnp.zeros((N_PAD, acc2.shape[1]), jnp.float32).at[:n].set(acc2)

    u, v = _norm_uv(acc2_pad, b2, Wl)
    u10 = u + bl[0]
    v10 = v
    nm = mask.shape[0]
    m_pad = ((nm + NW * 16 - 1) // (NW * 16)) * (NW * 16)
    m0 = jnp.pad(mask[:, 0], (0, m_pad - nm))
    m1 = jnp.pad(mask[:, 1], (0, m_pad - nm))
    p = _pair_score(m0, m1, u10, v10)
    return p[:nm, None]


# speedup vs baseline: 26.9202x; 2.2943x over previous
import functools

import jax
import jax.numpy as jnp
from jax import lax
from jax.experimental import pallas as pl
from jax.experimental.pallas import tpu as pltpu
from jax.experimental.pallas import tpu_sc as plsc

N_PAD = 10240
NC = 2
NS = 16
NW = NC * NS


def _pair_score_body(m0_hbm, m1_hbm, u_hbm, v_hbm, out_hbm,
                     m0_v, m1_v, u_v, v_v, o_v):
    wid = lax.axis_index("s") * NC + lax.axis_index("c")
    chunk = m0_v.shape[0]
    base = wid * chunk
    pltpu.sync_copy(u_hbm, u_v)
    pltpu.sync_copy(v_hbm, v_v)
    pltpu.sync_copy(m0_hbm.at[pl.ds(base, chunk)], m0_v)
    pltpu.sync_copy(m1_hbm.at[pl.ds(base, chunk)], m1_v)

    def vec(j, _):
        i0 = m0_v[pl.ds(j * 16, 16)]
        i1 = m1_v[pl.ds(j * 16, 16)]
        x = plsc.load_gather(u_v, [i0]) + plsc.load_gather(v_v, [i1])
        o_v[pl.ds(j * 16, 16)] = 1.0 / (1.0 + jnp.exp(-x))
        return _

    lax.fori_loop(0, chunk // 16, vec, 0)
    pltpu.sync_copy(o_v, out_hbm.at[pl.ds(base, chunk)])


def _pair_score(m0, m1, u, v):
    mp = m0.shape[0]
    n = u.shape[0]
    chunk = mp // NW
    mesh = plsc.VectorSubcoreMesh(core_axis_name="c", subcore_axis_name="s")
    f = pl.kernel(
        _pair_score_body,
        out_type=jax.ShapeDtypeStruct((mp,), jnp.float32),
        mesh=mesh,
        compiler_params=pltpu.CompilerParams(
            use_tc_tiling_on_sc=False, needs_layout_passes=False),
        scratch_types=[
            pltpu.VMEM((chunk,), jnp.int32),
            pltpu.VMEM((chunk,), jnp.int32),
            pltpu.VMEM((n,), jnp.float32),
            pltpu.VMEM((n,), jnp.float32),
            pltpu.VMEM((chunk,), jnp.float32),
        ],
    )
    return f(m0, m1, u, v)


HALF = 5000
ACC_R = 5008
DUMP = 5000
K = 48
BS = 2064
NA = 10016
SB = 14

_SC_PARAMS = dict(
    compiler_params=pltpu.CompilerParams(
        use_tc_tiling_on_sc=False, needs_layout_passes=False),
)


def _phase_a_body_factory(CHUNK, LISTSZ):
    def body(src_hbm, dst_hbm, asrc_hbm, adst_hbm, plist_hbm, slist_hbm,
             ebuf_s, ebuf_d, asrc_v, adst_v, lp, ls):
        c = lax.axis_index("c")
        sid = lax.axis_index("s")
        base = c * HALF
        pltpu.sync_copy(asrc_hbm, asrc_v)
        pltpu.sync_copy(adst_hbm, adst_v)
        chunk0 = sid * CHUNK

        def stage(b, cnt):
            eoff = chunk0 + b * BS
            pltpu.sync_copy(src_hbm.at[pl.ds(eoff, BS)], ebuf_s)
            pltpu.sync_copy(dst_hbm.at[pl.ds(eoff, BS)], ebuf_d)

            def vec(j, cnt):
                sv = ebuf_s[pl.ds(j * 16, 16)]
                dv = ebuf_d[pl.ds(j * 16, 16)]
                a = (plsc.load_gather(asrc_v, [sv])
                     + plsc.load_gather(adst_v, [dv]))
                a = jnp.where(a >= 0.0, a, a * jnp.float32(0.2))
                s = jnp.exp(a)
                own = (dv >= base) & (dv < base + HALF)
                inc = plsc.cumsum(own.astype(jnp.int32))
                pos = cnt + inc - 1
                packed = sv | ((dv - base) << SB)
                plsc.store_scatter(lp, [pos], packed, mask=own)
                plsc.store_scatter(ls, [pos], s, mask=own)
                return cnt + jnp.max(inc)

            return lax.fori_loop(0, BS // 16, vec, cnt)

        cnt = lax.fori_loop(0, CHUNK // BS, stage, jnp.int32(0))

        cntk = ((cnt + K - 1) // K) * K
        iota = lax.iota(jnp.int32, 16)
        dump = jnp.full((16,), DUMP << SB, jnp.int32)
        sent = jnp.full((16,), -1, jnp.int32)
        zero = jnp.zeros((16,), jnp.float32)
        for t in range(K // 16):
            pos = cnt + t * 16 + iota
            m = pos < cntk
            plsc.store_scatter(lp, [pos], dump, mask=m)
            plsc.store_scatter(ls, [pos], zero, mask=m)
        for t in range(K // 16):
            plsc.store_scatter(lp, [cntk + t * 16 + iota], sent)

        pltpu.sync_copy(lp, plist_hbm.at[c, sid])
        pltpu.sync_copy(ls, slist_hbm.at[c, sid])

    return body


def _phase_b_body_factory(D, LISTSZ):
    W = D + 16

    def body(plist_hbm, slist_hbm, h_hbm, out_hbm,
             pch0, src0, dst0, s0, gbuf0, sem0,
             pch1, src1, dst1, s1, gbuf1, sem1,
             sbuf, dstsc, scsem, acc):
        c = lax.axis_index("c")
        sid = lax.axis_index("s")

        def zrow(r, t):
            for cc in range(W // 16):
                sbuf[r, pl.ds(cc * 16, 16)] = jnp.zeros((16,), jnp.float32)
            return t
        lax.fori_loop(0, K, zrow, 0)
        stripe0 = sid * (ACC_R // NS)
        n_full = (ACC_R // NS) // K
        for t in range(n_full):
            pltpu.sync_copy(sbuf, acc.at[pl.ds(stripe0 + t * K, K)])
        rem = (ACC_R // NS) - n_full * K
        if rem:
            pltpu.sync_copy(sbuf.at[pl.ds(0, rem)],
                            acc.at[pl.ds(stripe0 + n_full * K, rem)])
        plsc.subcore_barrier()

        def peek(pch, i):
            pltpu.sync_copy(plist_hbm.at[c, sid, pl.ds(i * K, K)], pch)
            vs = [pch[pl.ds(q * 16, 16)] for q in range(K // 16)]
            mx = vs[0]
            for v in vs[1:]:
                mx = jnp.maximum(mx, v)
            return jnp.max(mx) >= 0, vs

        def prep(vs, src_ch, dst_ch, s_ch, gbuf, sem, i):
            pltpu.sync_copy(slist_hbm.at[c, sid, pl.ds(i * K, K)], s_ch)
            for q, vq in enumerate(vs):
                src_ch[pl.ds(q * 16, 16)] = vq & ((1 << SB) - 1)
                dst_ch[pl.ds(q * 16, 16)] = vq >> SB
            pltpu.async_copy(h_hbm.at[src_ch], gbuf, sem)

        def wait_scatter():
            pltpu.make_async_copy(sbuf, acc.at[dstsc], scsem).wait()

        def process(src_ch, dst_ch, s_ch, gbuf, sem):
            pltpu.make_async_copy(h_hbm.at[src_ch], gbuf, sem).wait()
            wait_scatter()

            @plsc.parallel_loop(0, K, 1, unroll=2)
            def row(r):
                sb = plsc.load_gather(s_ch, [lax.broadcast(r, (16,))])
                for cc in range(D // 16):
                    sbuf[r, pl.ds(cc * 16, 16)] = gbuf[r, pl.ds(cc * 16, 16)] * sb
                sbuf[r, pl.ds(D, 16)] = sb

            for q in range(K // 16):
                dstsc[pl.ds(q * 16, 16)] = dst_ch[pl.ds(q * 16, 16)]
            pltpu.async_copy(sbuf, acc.at[dstsc], scsem, add=True)

        dump_v = jnp.full((16,), DUMP, jnp.int32)
        for q in range(K // 16):
            dstsc[pl.ds(q * 16, 16)] = dump_v
        pltpu.async_copy(sbuf, acc.at[dstsc], scsem, add=True)

        gA, vs0 = peek(pch0, jnp.int32(0))

        @pl.when(gA)
        def _():
            prep(vs0, src0, dst0, s0, gbuf0, sem0, jnp.int32(0))
        gB_raw, vs1 = peek(pch1, jnp.int32(1))
        gB = gA & gB_raw

        @pl.when(gB)
        def _():
            prep(vs1, src1, dst1, s1, gbuf1, sem1, jnp.int32(1))

        def pair(st):
            i, ga, gb = st
            gc_raw, vsn0 = peek(pch0, i + 2)
            gc = gb & gc_raw
            process(src0, dst0, s0, gbuf0, sem0)

            @pl.when(gc)
            def _():
                prep(vsn0, src0, dst0, s0, gbuf0, sem0, i + 2)

            @pl.when(gb)
            def _():
                process(src1, dst1, s1, gbuf1, sem1)
            gd_raw, vsn1 = peek(pch1, i + 3)
            gd = gc & gd_raw

            @pl.when(gd)
            def _():
                prep(vsn1, src1, dst1, s1, gbuf1, sem1, i + 3)

            return i + 2, gc, gd

        lax.while_loop(lambda st: st[1], pair, (jnp.int32(0), gA, gB))
        wait_scatter()
        plsc.subcore_barrier()

        @pl.when(sid < NS - 1)
        def _():
            pltpu.sync_copy(acc.at[pl.ds(sid * 313, 313)],
                            out_hbm.at[c, pl.ds(sid * 313, 313)])

        @pl.when(sid == NS - 1)
        def _():
            pltpu.sync_copy(acc.at[pl.ds(15 * 313, HALF - 15 * 313)],
                            out_hbm.at[c, pl.ds(15 * 313, HALF - 15 * 313)])

    return body


def _edge_aggregate_sc(h, asrc, adst, src_pad, dst_pad):
    D = h.shape[1]
    W = D + 16
    E_PAD = src_pad.shape[0]
    CHUNK = E_PAD // NS
    LISTSZ = CHUNK + 3 * K
    mesh = plsc.VectorSubcoreMesh(core_axis_name="c", subcore_axis_name="s")

    fa = pl.kernel(
        _phase_a_body_factory(CHUNK, LISTSZ),
        out_type=[
            jax.ShapeDtypeStruct((NC, NS, LISTSZ), jnp.int32),
            jax.ShapeDtypeStruct((NC, NS, LISTSZ), jnp.float32),
        ],
        mesh=mesh,
        scratch_types=[
            pltpu.VMEM((BS,), jnp.int32),
            pltpu.VMEM((BS,), jnp.int32),
            pltpu.VMEM((NA,), jnp.float32),
            pltpu.VMEM((NA,), jnp.float32),
            pltpu.VMEM((LISTSZ,), jnp.int32),
            pltpu.VMEM((LISTSZ,), jnp.float32),
        ],
        **_SC_PARAMS,
    )
    plist, slist = fa(src_pad, dst_pad, asrc, adst)

    fb = pl.kernel(
        _phase_b_body_factory(D, LISTSZ),
        out_type=jax.ShapeDtypeStruct((NC, HALF, W), jnp.float32),
        mesh=mesh,
        scratch_types=[
            pltpu.VMEM((K,), jnp.int32),
            pltpu.VMEM((K,), jnp.int32),
            pltpu.VMEM((K,), jnp.int32),
            pltpu.VMEM((K,), jnp.float32),
            pltpu.VMEM((K, D), jnp.float32),
            pltpu.SemaphoreType.DMA,
            pltpu.VMEM((K,), jnp.int32),
            pltpu.VMEM((K,), jnp.int32),
            pltpu.VMEM((K,), jnp.int32),
            pltpu.VMEM((K,), jnp.float32),
            pltpu.VMEM((K, D), jnp.float32),
            pltpu.SemaphoreType.DMA,
            pltpu.VMEM((K, W), jnp.float32),
            pltpu.VMEM((K,), jnp.int32),
            pltpu.SemaphoreType.DMA,
            pltpu.VMEM_SHARED((ACC_R, W), jnp.float32),
        ],
        **_SC_PARAMS,
    )
    acc = fb(plist, slist, h)
    return acc.reshape(NC * HALF, W)


def _mm_attn_body(x_ref, w_ref, asrc_ref, adst_ref, h_ref, as_ref, ad_ref):
    h = x_ref[...] @ w_ref[...]
    h_ref[...] = h
    as_ref[...] = jnp.sum(h * asrc_ref[...], axis=-1)
    ad_ref[...] = jnp.sum(h * adst_ref[...], axis=-1)


def _mm_attn(x_pad, W, att_src, att_dst):
    n, d_in = x_pad.shape
    d_out = W.shape[1]
    blk = 1024
    grid = n // blk
    return pl.pallas_call(
        _mm_attn_body,
        grid=(grid,),
        in_specs=[
            pl.BlockSpec((blk, d_in), lambda i: (i, 0)),
            pl.BlockSpec((d_in, d_out), lambda i: (0, 0)),
            pl.BlockSpec((1, d_out), lambda i: (0, 0)),
            pl.BlockSpec((1, d_out), lambda i: (0, 0)),
        ],
        out_specs=[
            pl.BlockSpec((blk, d_out), lambda i: (i, 0)),
            pl.BlockSpec((blk,), lambda i: (i,)),
            pl.BlockSpec((blk,), lambda i: (i,)),
        ],
        out_shape=[
            jax.ShapeDtypeStruct((n, d_out), jnp.float32),
            jax.ShapeDtypeStruct((n,), jnp.float32),
            jax.ShapeDtypeStruct((n,), jnp.float32),
        ],
    )(x_pad, W, att_src.reshape(1, -1), att_dst.reshape(1, -1))


def _norm_mm_attn_body(acc_ref, b_ref, w_ref, asrc_ref, adst_ref,
                       h_ref, as_ref, ad_ref, *, d_prev):
    x = acc_ref[...]
    num = x[:, :d_prev]
    den = x[:, d_prev:d_prev + 1]
    h_in = jax.nn.relu(num / (den + 1e-16) + b_ref[...])
    h = h_in @ w_ref[...]
    h_ref[...] = h
    as_ref[...] = jnp.sum(h * asrc_ref[...], axis=-1)
    ad_ref[...] = jnp.sum(h * adst_ref[...], axis=-1)


def _norm_mm_attn(acc_pad, b, W, att_src, att_dst):
    n, w_in = acc_pad.shape
    d_prev = b.shape[0]
    d_out = W.shape[1]
    blk = 1024
    return pl.pallas_call(
        functools.partial(_norm_mm_attn_body, d_prev=d_prev),
        grid=(n // blk,),
        in_specs=[
            pl.BlockSpec((blk, w_in), lambda i: (i, 0)),
            pl.BlockSpec((1, d_prev), lambda i: (0, 0)),
            pl.BlockSpec((d_prev, d_out), lambda i: (0, 0)),
            pl.BlockSpec((1, d_out), lambda i: (0, 0)),
            pl.BlockSpec((1, d_out), lambda i: (0, 0)),
        ],
        out_specs=[
            pl.BlockSpec((blk, d_out), lambda i: (i, 0)),
            pl.BlockSpec((blk,), lambda i: (i,)),
            pl.BlockSpec((blk,), lambda i: (i,)),
        ],
        out_shape=[
            jax.ShapeDtypeStruct((n, d_out), jnp.float32),
            jax.ShapeDtypeStruct((n,), jnp.float32),
            jax.ShapeDtypeStruct((n,), jnp.float32),
        ],
    )(acc_pad, b.reshape(1, -1), W,
      att_src.reshape(1, -1), att_dst.reshape(1, -1))


def _norm_uv_body(acc_ref, b_ref, wlu_ref, wlv_ref, u_ref, v_ref, *, d_prev):
    x = acc_ref[...]
    num = x[:, :d_prev]
    den = x[:, d_prev:d_prev + 1]
    o = num / (den + 1e-16) + b_ref[...]
    u_ref[...] = jnp.sum(o * wlu_ref[...], axis=-1)
    v_ref[...] = jnp.sum(o * wlv_ref[...], axis=-1)


def _norm_uv(acc_pad, b, Wl):
    n, w_in = acc_pad.shape
    d_prev = b.shape[0]
    blk = 1024
    return pl.pallas_call(
        functools.partial(_norm_uv_body, d_prev=d_prev),
        grid=(n // blk,),
        in_specs=[
            pl.BlockSpec((blk, w_in), lambda i: (i, 0)),
            pl.BlockSpec((1, d_prev), lambda i: (0, 0)),
            pl.BlockSpec((1, d_prev), lambda i: (0, 0)),
            pl.BlockSpec((1, d_prev), lambda i: (0, 0)),
        ],
        out_specs=[
            pl.BlockSpec((blk,), lambda i: (i,)),
            pl.BlockSpec((blk,), lambda i: (i,)),
        ],
        out_shape=[
            jax.ShapeDtypeStruct((n,), jnp.float32),
            jax.ShapeDtypeStruct((n,), jnp.float32),
        ],
    )(acc_pad, b.reshape(1, -1),
      Wl[:d_prev, 0].reshape(1, -1), Wl[d_prev:, 0].reshape(1, -1))


def kernel(g, features, mask, W1, att_src1, att_dst1, b1,
           W2, att_src2, att_dst2, b2, Wl, bl):
    n = features.shape[0]
    e = g.shape[1]
    e_pad = ((e + n + NS * BS - 1) // (NS * BS)) * (NS * BS)
    loop = jnp.arange(n, dtype=g.dtype)
    src_pad = jnp.concatenate(
        [g[0], loop, jnp.zeros((e_pad - e - n,), g.dtype)])
    dst_pad = jnp.concatenate(
        [g[1], loop, jnp.full((e_pad - e - n,), n, g.dtype)])

    x_pad = jnp.zeros((N_PAD, features.shape[1]), jnp.float32).at[:n].set(features)

    h1, as1, ad1 = _mm_attn(x_pad, W1, att_src1, att_dst1)
    acc1 = _edge_aggregate_sc(h1, as1[:NA], ad1[:NA], src_pad, dst_pad)
    acc1_pad = jnp.zeros((N_PAD, acc1.shape[1]), jnp.float32).at[:n].set(acc1)

    h2, as2, ad2 = _norm_mm_attn(acc1_pad, b1, W2, att_src2, att_dst2)
    acc2 = _edge_aggregate_sc(h2, as2[:NA], ad2[:NA], src_pad, dst_pad)
    acc2_pad = jnp.zeros((N_PAD, acc2.shape[1]), jnp.float32).at[:n].set(acc2)

    u, v = _norm_uv(acc2_pad, b2, Wl)
    u10 = u + bl[0]
    v10 = v
    nm = mask.shape[0]
    m_pad = ((nm + NW * 16 - 1) // (NW * 16)) * (NW * 16)
    m0 = jnp.pad(mask[:, 0], (0, m_pad - nm))
    m1 = jnp.pad(mask[:, 1], (0, m_pad - nm))
    p = _pair_score(m0, m1, u10, v10)
    return p[:nm, None]


# speedup vs baseline: 32.2828x; 1.1992x over previous
import functools

import jax
import jax.numpy as jnp
from jax import lax
from jax.experimental import pallas as pl
from jax.experimental.pallas import tpu as pltpu
from jax.experimental.pallas import tpu_sc as plsc

N_PAD = 10240
NC = 2
NS = 16
NW = NC * NS


def _pair_score_body(m0_hbm, m1_hbm, u_hbm, v_hbm, out_hbm,
                     m0_v, m1_v, u_v, v_v, o_v):
    wid = lax.axis_index("s") * NC + lax.axis_index("c")
    chunk = m0_v.shape[0]
    base = wid * chunk
    pltpu.sync_copy(u_hbm, u_v)
    pltpu.sync_copy(v_hbm, v_v)
    pltpu.sync_copy(m0_hbm.at[pl.ds(base, chunk)], m0_v)
    pltpu.sync_copy(m1_hbm.at[pl.ds(base, chunk)], m1_v)

    def vec(j, _):
        i0 = m0_v[pl.ds(j * 16, 16)]
        i1 = m1_v[pl.ds(j * 16, 16)]
        x = plsc.load_gather(u_v, [i0]) + plsc.load_gather(v_v, [i1])
        o_v[pl.ds(j * 16, 16)] = 1.0 / (1.0 + jnp.exp(-x))
        return _

    lax.fori_loop(0, chunk // 16, vec, 0)
    pltpu.sync_copy(o_v, out_hbm.at[pl.ds(base, chunk)])


def _pair_score(m0, m1, u, v):
    mp = m0.shape[0]
    n = u.shape[0]
    chunk = mp // NW
    mesh = plsc.VectorSubcoreMesh(core_axis_name="c", subcore_axis_name="s")
    f = pl.kernel(
        _pair_score_body,
        out_type=jax.ShapeDtypeStruct((mp,), jnp.float32),
        mesh=mesh,
        compiler_params=pltpu.CompilerParams(
            use_tc_tiling_on_sc=False, needs_layout_passes=False),
        scratch_types=[
            pltpu.VMEM((chunk,), jnp.int32),
            pltpu.VMEM((chunk,), jnp.int32),
            pltpu.VMEM((n,), jnp.float32),
            pltpu.VMEM((n,), jnp.float32),
            pltpu.VMEM((chunk,), jnp.float32),
        ],
    )
    return f(m0, m1, u, v)


HALF = 5000
ACC_R = 5008
DUMP = 5000
K = 48
BS = 2064
NA = 10016
SB = 14

_SC_PARAMS = dict(
    compiler_params=pltpu.CompilerParams(
        use_tc_tiling_on_sc=False, needs_layout_passes=False),
)


def _phase_a_body_factory(CHUNK, LISTSZ):
    def body(src_hbm, dst_hbm, asrc_hbm, adst_hbm, clist_hbm,
             ebuf_s0, ebuf_d0, ebuf_s1, ebuf_d1, asrc_v, adst_v, cl,
             sem0, sem1):
        c = lax.axis_index("c")
        sid = lax.axis_index("s")
        base = c * HALF
        pltpu.sync_copy(asrc_hbm, asrc_v)
        pltpu.sync_copy(adst_hbm, adst_v)
        chunk0 = sid * CHUNK
        nstage = CHUNK // BS

        def start(b, ebuf_s, ebuf_d, sem):
            eoff = chunk0 + b * BS
            pltpu.async_copy(src_hbm.at[pl.ds(eoff, BS)], ebuf_s, sem)
            pltpu.async_copy(dst_hbm.at[pl.ds(eoff, BS)], ebuf_d, sem)

        def wait(b, ebuf_s, ebuf_d, sem):
            eoff = chunk0 + b * BS
            pltpu.make_async_copy(src_hbm.at[pl.ds(eoff, BS)], ebuf_s, sem).wait()
            pltpu.make_async_copy(dst_hbm.at[pl.ds(eoff, BS)], ebuf_d, sem).wait()

        def scan(ebuf_s, ebuf_d, cnt0):
            @plsc.parallel_loop(0, BS // 16, 1, unroll=2, carry=cnt0)
            def vec(j, cnt):
                sv = ebuf_s[pl.ds(j * 16, 16)]
                dv = ebuf_d[pl.ds(j * 16, 16)]
                a = (plsc.load_gather(asrc_v, [sv])
                     + plsc.load_gather(adst_v, [dv]))
                a = jnp.where(a >= 0.0, a, a * jnp.float32(0.2))
                s = jnp.exp(a)
                own = (dv >= base) & (dv < base + HALF)
                inc = plsc.cumsum(own.astype(jnp.int32))
                pos = cnt + inc - 1
                packed = sv | ((dv - base) << SB)
                plsc.store_scatter(cl, [jnp.zeros((16,), jnp.int32), pos],
                                   packed, mask=own)
                plsc.store_scatter(cl, [jnp.ones((16,), jnp.int32), pos],
                                   plsc.bitcast(s, jnp.int32), mask=own)
                return cnt + jnp.max(inc)
            return vec

        assert nstage % 2 == 0
        start(0, ebuf_s0, ebuf_d0, sem0)

        def stage2(b2, cnt):
            b = b2 * 2
            wait(b, ebuf_s0, ebuf_d0, sem0)
            start(b + 1, ebuf_s1, ebuf_d1, sem1)
            cnt = scan(ebuf_s0, ebuf_d0, cnt)
            wait(b + 1, ebuf_s1, ebuf_d1, sem1)

            @pl.when(b + 2 < nstage)
            def _():
                start(b + 2, ebuf_s0, ebuf_d0, sem0)
            return scan(ebuf_s1, ebuf_d1, cnt)

        cnt = lax.fori_loop(0, nstage // 2, stage2, jnp.int32(0))

        cntk = ((cnt + K - 1) // K) * K
        iota = lax.iota(jnp.int32, 16)
        dump = jnp.full((16,), DUMP << SB, jnp.int32)
        sent = jnp.full((16,), -1, jnp.int32)
        zero = jnp.zeros((16,), jnp.int32)
        ones = jnp.ones((16,), jnp.int32)
        for t in range(K // 16):
            pos = cnt + t * 16 + iota
            m = pos < cntk
            plsc.store_scatter(cl, [zero, pos], dump, mask=m)
            plsc.store_scatter(cl, [ones, pos], zero, mask=m)
        for t in range(K // 16):
            plsc.store_scatter(cl, [zero, cntk + t * 16 + iota], sent)

        pltpu.sync_copy(cl, clist_hbm.at[c, sid])

    return body


def _phase_b_body_factory(D, LISTSZ):
    W = D + 16

    def body(clist_hbm, h_hbm, out_hbm,
             cb0, src0, dst0, s0, gbuf0, sem0,
             cb1, src1, dst1, s1, gbuf1, sem1,
             sbuf, dstsc, scsem, acc):
        c = lax.axis_index("c")
        sid = lax.axis_index("s")

        def zrow(r, t):
            for cc in range(W // 16):
                sbuf[r, pl.ds(cc * 16, 16)] = jnp.zeros((16,), jnp.float32)
            return t
        lax.fori_loop(0, K, zrow, 0)
        stripe0 = sid * (ACC_R // NS)
        n_full = (ACC_R // NS) // K
        for t in range(n_full):
            pltpu.sync_copy(sbuf, acc.at[pl.ds(stripe0 + t * K, K)])
        rem = (ACC_R // NS) - n_full * K
        if rem:
            pltpu.sync_copy(sbuf.at[pl.ds(0, rem)],
                            acc.at[pl.ds(stripe0 + n_full * K, rem)])
        plsc.subcore_barrier()

        def peek(cb, i):
            pltpu.sync_copy(clist_hbm.at[c, sid, :, pl.ds(i * K, K)], cb)
            vs = [cb[0, pl.ds(q * 16, 16)] for q in range(K // 16)]
            mx = vs[0]
            for v in vs[1:]:
                mx = jnp.maximum(mx, v)
            return jnp.max(mx) >= 0, vs

        def prep(vs, cb, src_ch, dst_ch, s_ch, gbuf, sem, i):
            for q, vq in enumerate(vs):
                src_ch[pl.ds(q * 16, 16)] = vq & ((1 << SB) - 1)
                dst_ch[pl.ds(q * 16, 16)] = vq >> SB
                s_ch[pl.ds(q * 16, 16)] = plsc.bitcast(
                    cb[1, pl.ds(q * 16, 16)], jnp.float32)
            pltpu.async_copy(h_hbm.at[src_ch], gbuf, sem)

        def wait_scatter():
            pltpu.make_async_copy(sbuf, acc.at[dstsc], scsem).wait()

        def process(src_ch, dst_ch, s_ch, gbuf, sem):
            pltpu.make_async_copy(h_hbm.at[src_ch], gbuf, sem).wait()
            wait_scatter()

            @plsc.parallel_loop(0, K, 1, unroll=2)
            def row(r):
                sb = plsc.load_gather(s_ch, [lax.broadcast(r, (16,))])
                for cc in range(D // 16):
                    sbuf[r, pl.ds(cc * 16, 16)] = gbuf[r, pl.ds(cc * 16, 16)] * sb
                sbuf[r, pl.ds(D, 16)] = sb

            for q in range(K // 16):
                dstsc[pl.ds(q * 16, 16)] = dst_ch[pl.ds(q * 16, 16)]
            pltpu.async_copy(sbuf, acc.at[dstsc], scsem, add=True)

        dump_v = jnp.full((16,), DUMP, jnp.int32)
        for q in range(K // 16):
            dstsc[pl.ds(q * 16, 16)] = dump_v
        pltpu.async_copy(sbuf, acc.at[dstsc], scsem, add=True)

        gA, vs0 = peek(cb0, jnp.int32(0))

        @pl.when(gA)
        def _():
            prep(vs0, cb0, src0, dst0, s0, gbuf0, sem0, jnp.int32(0))
        gB_raw, vs1 = peek(cb1, jnp.int32(1))
        gB = gA & gB_raw

        @pl.when(gB)
        def _():
            prep(vs1, cb1, src1, dst1, s1, gbuf1, sem1, jnp.int32(1))

        def pair(st):
            i, ga, gb = st
            gc_raw, vsn0 = peek(cb0, i + 2)
            gc = gb & gc_raw
            process(src0, dst0, s0, gbuf0, sem0)

            @pl.when(gc)
            def _():
                prep(vsn0, cb0, src0, dst0, s0, gbuf0, sem0, i + 2)

            @pl.when(gb)
            def _():
                process(src1, dst1, s1, gbuf1, sem1)
            gd_raw, vsn1 = peek(cb1, i + 3)
            gd = gc & gd_raw

            @pl.when(gd)
            def _():
                prep(vsn1, cb1, src1, dst1, s1, gbuf1, sem1, i + 3)

            return i + 2, gc, gd

        lax.while_loop(lambda st: st[1], pair, (jnp.int32(0), gA, gB))
        wait_scatter()
        plsc.subcore_barrier()

        @pl.when(sid < NS - 1)
        def _():
            pltpu.sync_copy(acc.at[pl.ds(sid * 313, 313)],
                            out_hbm.at[c, pl.ds(sid * 313, 313)])

        @pl.when(sid == NS - 1)
        def _():
            pltpu.sync_copy(acc.at[pl.ds(15 * 313, HALF - 15 * 313)],
                            out_hbm.at[c, pl.ds(15 * 313, HALF - 15 * 313)])

    return body


def _edge_aggregate_sc(h, asrc, adst, src_pad, dst_pad):
    D = h.shape[1]
    W = D + 16
    E_PAD = src_pad.shape[0]
    CHUNK = E_PAD // NS
    LISTSZ = CHUNK + 3 * K
    mesh = plsc.VectorSubcoreMesh(core_axis_name="c", subcore_axis_name="s")

    fa = pl.kernel(
        _phase_a_body_factory(CHUNK, LISTSZ),
        out_type=jax.ShapeDtypeStruct((NC, NS, 2, LISTSZ), jnp.int32),
        mesh=mesh,
        scratch_types=[
            pltpu.VMEM((BS,), jnp.int32),
            pltpu.VMEM((BS,), jnp.int32),
            pltpu.VMEM((BS,), jnp.int32),
            pltpu.VMEM((BS,), jnp.int32),
            pltpu.VMEM((NA,), jnp.float32),
            pltpu.VMEM((NA,), jnp.float32),
            pltpu.VMEM((2, LISTSZ), jnp.int32),
            pltpu.SemaphoreType.DMA,
            pltpu.SemaphoreType.DMA,
        ],
        **_SC_PARAMS,
    )
    clist = fa(src_pad, dst_pad, asrc, adst)

    fb = pl.kernel(
        _phase_b_body_factory(D, LISTSZ),
        out_type=jax.ShapeDtypeStruct((NC, HALF, W), jnp.float32),
        mesh=mesh,
        scratch_types=[
            pltpu.VMEM((2, K), jnp.int32),
            pltpu.VMEM((K,), jnp.int32),
            pltpu.VMEM((K,), jnp.int32),
            pltpu.VMEM((K,), jnp.float32),
            pltpu.VMEM((K, D), jnp.float32),
            pltpu.SemaphoreType.DMA,
            pltpu.VMEM((2, K), jnp.int32),
            pltpu.VMEM((K,), jnp.int32),
            pltpu.VMEM((K,), jnp.int32),
            pltpu.VMEM((K,), jnp.float32),
            pltpu.VMEM((K, D), jnp.float32),
            pltpu.SemaphoreType.DMA,
            pltpu.VMEM((K, W), jnp.float32),
            pltpu.VMEM((K,), jnp.int32),
            pltpu.SemaphoreType.DMA,
            pltpu.VMEM_SHARED((ACC_R, W), jnp.float32),
        ],
        **_SC_PARAMS,
    )
    acc = fb(clist, h)
    return acc.reshape(NC * HALF, W)


def _mm_attn_body(x_ref, w_ref, asrc_ref, adst_ref, h_ref, as_ref, ad_ref):
    h = x_ref[...] @ w_ref[...]
    h_ref[...] = h
    as_ref[...] = jnp.sum(h * asrc_ref[...], axis=-1)
    ad_ref[...] = jnp.sum(h * adst_ref[...], axis=-1)


def _mm_attn(x_pad, W, att_src, att_dst):
    n, d_in = x_pad.shape
    d_out = W.shape[1]
    blk = 1024
    grid = n // blk
    return pl.pallas_call(
        _mm_attn_body,
        grid=(grid,),
        in_specs=[
            pl.BlockSpec((blk, d_in), lambda i: (i, 0)),
            pl.BlockSpec((d_in, d_out), lambda i: (0, 0)),
            pl.BlockSpec((1, d_out), lambda i: (0, 0)),
            pl.BlockSpec((1, d_out), lambda i: (0, 0)),
        ],
        out_specs=[
            pl.BlockSpec((blk, d_out), lambda i: (i, 0)),
            pl.BlockSpec((blk,), lambda i: (i,)),
            pl.BlockSpec((blk,), lambda i: (i,)),
        ],
        out_shape=[
            jax.ShapeDtypeStruct((n, d_out), jnp.float32),
            jax.ShapeDtypeStruct((n,), jnp.float32),
            jax.ShapeDtypeStruct((n,), jnp.float32),
        ],
    )(x_pad, W, att_src.reshape(1, -1), att_dst.reshape(1, -1))


def _norm_mm_attn_body(acc_ref, b_ref, w_ref, asrc_ref, adst_ref,
                       h_ref, as_ref, ad_ref, *, d_prev):
    x = acc_ref[...]
    num = x[:, :d_prev]
    den = x[:, d_prev:d_prev + 1]
    h_in = jax.nn.relu(num / (den + 1e-16) + b_ref[...])
    h = h_in @ w_ref[...]
    h_ref[...] = h
    as_ref[...] = jnp.sum(h * asrc_ref[...], axis=-1)
    ad_ref[...] = jnp.sum(h * adst_ref[...], axis=-1)


def _norm_mm_attn(acc_pad, b, W, att_src, att_dst):
    n, w_in = acc_pad.shape
    d_prev = b.shape[0]
    d_out = W.shape[1]
    blk = 1024
    return pl.pallas_call(
        functools.partial(_norm_mm_attn_body, d_prev=d_prev),
        grid=(n // blk,),
        in_specs=[
            pl.BlockSpec((blk, w_in), lambda i: (i, 0)),
            pl.BlockSpec((1, d_prev), lambda i: (0, 0)),
            pl.BlockSpec((d_prev, d_out), lambda i: (0, 0)),
            pl.BlockSpec((1, d_out), lambda i: (0, 0)),
            pl.BlockSpec((1, d_out), lambda i: (0, 0)),
        ],
        out_specs=[
            pl.BlockSpec((blk, d_out), lambda i: (i, 0)),
            pl.BlockSpec((blk,), lambda i: (i,)),
            pl.BlockSpec((blk,), lambda i: (i,)),
        ],
        out_shape=[
            jax.ShapeDtypeStruct((n, d_out), jnp.float32),
            jax.ShapeDtypeStruct((n,), jnp.float32),
            jax.ShapeDtypeStruct((n,), jnp.float32),
        ],
    )(acc_pad, b.reshape(1, -1), W,
      att_src.reshape(1, -1), att_dst.reshape(1, -1))


def _norm_uv_body(acc_ref, b_ref, wlu_ref, wlv_ref, u_ref, v_ref, *, d_prev):
    x = acc_ref[...]
    num = x[:, :d_prev]
    den = x[:, d_prev:d_prev + 1]
    o = num / (den + 1e-16) + b_ref[...]
    u_ref[...] = jnp.sum(o * wlu_ref[...], axis=-1)
    v_ref[...] = jnp.sum(o * wlv_ref[...], axis=-1)


def _norm_uv(acc_pad, b, Wl):
    n, w_in = acc_pad.shape
    d_prev = b.shape[0]
    blk = 1024
    return pl.pallas_call(
        functools.partial(_norm_uv_body, d_prev=d_prev),
        grid=(n // blk,),
        in_specs=[
            pl.BlockSpec((blk, w_in), lambda i: (i, 0)),
            pl.BlockSpec((1, d_prev), lambda i: (0, 0)),
            pl.BlockSpec((1, d_prev), lambda i: (0, 0)),
            pl.BlockSpec((1, d_prev), lambda i: (0, 0)),
        ],
        out_specs=[
            pl.BlockSpec((blk,), lambda i: (i,)),
            pl.BlockSpec((blk,), lambda i: (i,)),
        ],
        out_shape=[
            jax.ShapeDtypeStruct((n,), jnp.float32),
            jax.ShapeDtypeStruct((n,), jnp.float32),
        ],
    )(acc_pad, b.reshape(1, -1),
      Wl[:d_prev, 0].reshape(1, -1), Wl[d_prev:, 0].reshape(1, -1))


def kernel(g, features, mask, W1, att_src1, att_dst1, b1,
           W2, att_src2, att_dst2, b2, Wl, bl):
    n = features.shape[0]
    e = g.shape[1]
    e_pad = ((e + n + NS * BS - 1) // (NS * BS)) * (NS * BS)
    loop = jnp.arange(n, dtype=g.dtype)
    src_pad = jnp.concatenate(
        [g[0], loop, jnp.zeros((e_pad - e - n,), g.dtype)])
    dst_pad = jnp.concatenate(
        [g[1], loop, jnp.full((e_pad - e - n,), n, g.dtype)])

    x_pad = jnp.zeros((N_PAD, features.shape[1]), jnp.float32).at[:n].set(features)

    h1, as1, ad1 = _mm_attn(x_pad, W1, att_src1, att_dst1)
    acc1 = _edge_aggregate_sc(h1, as1[:NA], ad1[:NA], src_pad, dst_pad)
    acc1_pad = jnp.zeros((N_PAD, acc1.shape[1]), jnp.float32).at[:n].set(acc1)

    h2, as2, ad2 = _norm_mm_attn(acc1_pad, b1, W2, att_src2, att_dst2)
    acc2 = _edge_aggregate_sc(h2, as2[:NA], ad2[:NA], src_pad, dst_pad)
    acc2_pad = jnp.zeros((N_PAD, acc2.shape[1]), jnp.float32).at[:n].set(acc2)

    u, v = _norm_uv(acc2_pad, b2, Wl)
    u10 = u + bl[0]
    v10 = v
    nm = mask.shape[0]
    m_pad = ((nm + NW * 16 - 1) // (NW * 16)) * (NW * 16)
    m0 = jnp.pad(mask[:, 0], (0, m_pad - nm))
    m1 = jnp.pad(mask[:, 1], (0, m_pad - nm))
    p = _pair_score(m0, m1, u10, v10)
    return p[:nm, None]


# speedup vs baseline: 32.5641x; 1.0087x over previous
import functools

import jax
import jax.numpy as jnp
from jax import lax
from jax.experimental import pallas as pl
from jax.experimental.pallas import tpu as pltpu
from jax.experimental.pallas import tpu_sc as plsc

N_PAD = 10240
NC = 2
NS = 16
NW = NC * NS


def _pair_score_body(m0_hbm, m1_hbm, u_hbm, v_hbm, out_hbm,
                     m0_v, m1_v, u_v, v_v, o_v):
    wid = lax.axis_index("s") * NC + lax.axis_index("c")
    chunk = m0_v.shape[0]
    base = wid * chunk
    pltpu.sync_copy(u_hbm, u_v)
    pltpu.sync_copy(v_hbm, v_v)
    pltpu.sync_copy(m0_hbm.at[pl.ds(base, chunk)], m0_v)
    pltpu.sync_copy(m1_hbm.at[pl.ds(base, chunk)], m1_v)

    def vec(j, _):
        i0 = m0_v[pl.ds(j * 16, 16)]
        i1 = m1_v[pl.ds(j * 16, 16)]
        x = plsc.load_gather(u_v, [i0]) + plsc.load_gather(v_v, [i1])
        o_v[pl.ds(j * 16, 16)] = 1.0 / (1.0 + jnp.exp(-x))
        return _

    lax.fori_loop(0, chunk // 16, vec, 0)
    pltpu.sync_copy(o_v, out_hbm.at[pl.ds(base, chunk)])


def _pair_score(m0, m1, u, v):
    mp = m0.shape[0]
    n = u.shape[0]
    chunk = mp // NW
    mesh = plsc.VectorSubcoreMesh(core_axis_name="c", subcore_axis_name="s")
    f = pl.kernel(
        _pair_score_body,
        out_type=jax.ShapeDtypeStruct((mp,), jnp.float32),
        mesh=mesh,
        compiler_params=pltpu.CompilerParams(
            use_tc_tiling_on_sc=False, needs_layout_passes=False),
        scratch_types=[
            pltpu.VMEM((chunk,), jnp.int32),
            pltpu.VMEM((chunk,), jnp.int32),
            pltpu.VMEM((n,), jnp.float32),
            pltpu.VMEM((n,), jnp.float32),
            pltpu.VMEM((chunk,), jnp.float32),
        ],
    )
    return f(m0, m1, u, v)


HALF = 5000
ACC_R = 5008
DUMP = 5000
K = 48
BS = 2064
NA = 10016
SB = 14

_SC_PARAMS = dict(
    compiler_params=pltpu.CompilerParams(
        use_tc_tiling_on_sc=False, needs_layout_passes=False),
)


def _phase_a_body_factory(CHUNK, LISTSZ):
    def body(src_hbm, dst_hbm, asrc_hbm, adst_hbm, clist_hbm,
             ebuf_s0, ebuf_d0, ebuf_s1, ebuf_d1, asrc_v, adst_v, cl,
             sem0, sem1):
        c = lax.axis_index("c")
        sid = lax.axis_index("s")
        base = c * HALF
        pltpu.sync_copy(asrc_hbm, asrc_v)
        pltpu.sync_copy(adst_hbm, adst_v)
        chunk0 = sid * CHUNK
        nstage = CHUNK // BS

        def start(b, ebuf_s, ebuf_d, sem):
            eoff = chunk0 + b * BS
            pltpu.async_copy(src_hbm.at[pl.ds(eoff, BS)], ebuf_s, sem)
            pltpu.async_copy(dst_hbm.at[pl.ds(eoff, BS)], ebuf_d, sem)

        def wait(b, ebuf_s, ebuf_d, sem):
            eoff = chunk0 + b * BS
            pltpu.make_async_copy(src_hbm.at[pl.ds(eoff, BS)], ebuf_s, sem).wait()
            pltpu.make_async_copy(dst_hbm.at[pl.ds(eoff, BS)], ebuf_d, sem).wait()

        def scan(ebuf_s, ebuf_d, cnt0):
            @plsc.parallel_loop(0, BS // 16, 1, unroll=2, carry=cnt0)
            def vec(j, cnt):
                sv = ebuf_s[pl.ds(j * 16, 16)]
                dv = ebuf_d[pl.ds(j * 16, 16)]
                a = (plsc.load_gather(asrc_v, [sv])
                     + plsc.load_gather(adst_v, [dv]))
                a = jnp.where(a >= 0.0, a, a * jnp.float32(0.2))
                s = jnp.exp(a)
                own = (dv >= base) & (dv < base + HALF)
                inc = plsc.cumsum(own.astype(jnp.int32))
                pos = cnt + inc - 1
                packed = sv | ((dv - base) << SB)
                plsc.store_scatter(cl, [jnp.zeros((16,), jnp.int32), pos],
                                   packed, mask=own)
                plsc.store_scatter(cl, [jnp.ones((16,), jnp.int32), pos],
                                   plsc.bitcast(s, jnp.int32), mask=own)
                return cnt + jnp.max(inc)
            return vec

        assert nstage % 2 == 0
        start(0, ebuf_s0, ebuf_d0, sem0)

        def stage2(b2, cnt):
            b = b2 * 2
            wait(b, ebuf_s0, ebuf_d0, sem0)
            start(b + 1, ebuf_s1, ebuf_d1, sem1)
            cnt = scan(ebuf_s0, ebuf_d0, cnt)
            wait(b + 1, ebuf_s1, ebuf_d1, sem1)

            @pl.when(b + 2 < nstage)
            def _():
                start(b + 2, ebuf_s0, ebuf_d0, sem0)
            return scan(ebuf_s1, ebuf_d1, cnt)

        cnt = lax.fori_loop(0, nstage // 2, stage2, jnp.int32(0))

        cntk = ((cnt + K - 1) // K) * K
        iota = lax.iota(jnp.int32, 16)
        dump = jnp.full((16,), DUMP << SB, jnp.int32)
        sent = jnp.full((16,), -1, jnp.int32)
        zero = jnp.zeros((16,), jnp.int32)
        ones = jnp.ones((16,), jnp.int32)
        for t in range(K // 16):
            pos = cnt + t * 16 + iota
            m = pos < cntk
            plsc.store_scatter(cl, [zero, pos], dump, mask=m)
            plsc.store_scatter(cl, [ones, pos], zero, mask=m)
        for t in range(K // 16):
            plsc.store_scatter(cl, [zero, cntk + t * 16 + iota], sent)

        pltpu.sync_copy(cl, clist_hbm.at[c, sid])

    return body


def _phase_b_body_factory(D, LISTSZ):
    W = D + 16

    def body(clist_hbm, h_hbm, out_hbm,
             cb0, src0, dst0, s0, gbuf0, sem0,
             cb1, src1, dst1, s1, gbuf1, sem1,
             sbuf, dstsc, scsem, acc):
        c = lax.axis_index("c")
        sid = lax.axis_index("s")

        def zrow(r, t):
            for cc in range(W // 16):
                sbuf[r, pl.ds(cc * 16, 16)] = jnp.zeros((16,), jnp.float32)
            return t
        lax.fori_loop(0, K, zrow, 0)
        stripe0 = sid * (ACC_R // NS)
        n_full = (ACC_R // NS) // K
        for t in range(n_full):
            pltpu.sync_copy(sbuf, acc.at[pl.ds(stripe0 + t * K, K)])
        rem = (ACC_R // NS) - n_full * K
        if rem:
            pltpu.sync_copy(sbuf.at[pl.ds(0, rem)],
                            acc.at[pl.ds(stripe0 + n_full * K, rem)])
        plsc.subcore_barrier()

        def peek(cb, i):
            pltpu.sync_copy(clist_hbm.at[c, sid, :, pl.ds(i * K, K)], cb)
            vs = [cb[0, pl.ds(q * 16, 16)] for q in range(K // 16)]
            mx = vs[0]
            for v in vs[1:]:
                mx = jnp.maximum(mx, v)
            return jnp.max(mx) >= 0, vs

        def prep(vs, cb, src_ch, dst_ch, s_ch, gbuf, sem, i):
            for q, vq in enumerate(vs):
                src_ch[pl.ds(q * 16, 16)] = vq & ((1 << SB) - 1)
                dst_ch[pl.ds(q * 16, 16)] = vq >> SB
                s_ch[pl.ds(q * 16, 16)] = plsc.bitcast(
                    cb[1, pl.ds(q * 16, 16)], jnp.float32)
            pltpu.async_copy(h_hbm.at[src_ch], gbuf, sem)

        def wait_scatter():
            pltpu.make_async_copy(sbuf, acc.at[dstsc], scsem).wait()

        def process(src_ch, dst_ch, s_ch, gbuf, sem):
            pltpu.make_async_copy(h_hbm.at[src_ch], gbuf, sem).wait()
            wait_scatter()

            @plsc.parallel_loop(0, K, 1, unroll=4)
            def row(r):
                sb = plsc.load_gather(s_ch, [lax.broadcast(r, (16,))])
                for cc in range(D // 16):
                    sbuf[r, pl.ds(cc * 16, 16)] = gbuf[r, pl.ds(cc * 16, 16)] * sb
                sbuf[r, pl.ds(D, 16)] = sb

            for q in range(K // 16):
                dstsc[pl.ds(q * 16, 16)] = dst_ch[pl.ds(q * 16, 16)]
            pltpu.async_copy(sbuf, acc.at[dstsc], scsem, add=True)

        dump_v = jnp.full((16,), DUMP, jnp.int32)
        for q in range(K // 16):
            dstsc[pl.ds(q * 16, 16)] = dump_v
        pltpu.async_copy(sbuf, acc.at[dstsc], scsem, add=True)

        gA, vs0 = peek(cb0, jnp.int32(0))

        @pl.when(gA)
        def _():
            prep(vs0, cb0, src0, dst0, s0, gbuf0, sem0, jnp.int32(0))
        gB_raw, vs1 = peek(cb1, jnp.int32(1))
        gB = gA & gB_raw

        @pl.when(gB)
        def _():
            prep(vs1, cb1, src1, dst1, s1, gbuf1, sem1, jnp.int32(1))

        def pair(st):
            i, ga, gb = st
            gc_raw, vsn0 = peek(cb0, i + 2)
            gc = gb & gc_raw
            process(src0, dst0, s0, gbuf0, sem0)

            @pl.when(gc)
            def _():
                prep(vsn0, cb0, src0, dst0, s0, gbuf0, sem0, i + 2)

            @pl.when(gb)
            def _():
                process(src1, dst1, s1, gbuf1, sem1)
            gd_raw, vsn1 = peek(cb1, i + 3)
            gd = gc & gd_raw

            @pl.when(gd)
            def _():
                prep(vsn1, cb1, src1, dst1, s1, gbuf1, sem1, i + 3)

            return i + 2, gc, gd

        lax.while_loop(lambda st: st[1], pair, (jnp.int32(0), gA, gB))
        wait_scatter()
        plsc.subcore_barrier()

        @pl.when(sid < NS - 1)
        def _():
            pltpu.sync_copy(acc.at[pl.ds(sid * 313, 313)],
                            out_hbm.at[c, pl.ds(sid * 313, 313)])

        @pl.when(sid == NS - 1)
        def _():
            pltpu.sync_copy(acc.at[pl.ds(15 * 313, HALF - 15 * 313)],
                            out_hbm.at[c, pl.ds(15 * 313, HALF - 15 * 313)])

    return body


def _edge_aggregate_sc(h, asrc, adst, src_pad, dst_pad):
    D = h.shape[1]
    W = D + 16
    E_PAD = src_pad.shape[0]
    CHUNK = E_PAD // NS
    LISTSZ = CHUNK + 3 * K
    mesh = plsc.VectorSubcoreMesh(core_axis_name="c", subcore_axis_name="s")

    fa = pl.kernel(
        _phase_a_body_factory(CHUNK, LISTSZ),
        out_type=jax.ShapeDtypeStruct((NC, NS, 2, LISTSZ), jnp.int32),
        mesh=mesh,
        scratch_types=[
            pltpu.VMEM((BS,), jnp.int32),
            pltpu.VMEM((BS,), jnp.int32),
            pltpu.VMEM((BS,), jnp.int32),
            pltpu.VMEM((BS,), jnp.int32),
            pltpu.VMEM((NA,), jnp.float32),
            pltpu.VMEM((NA,), jnp.float32),
            pltpu.VMEM((2, LISTSZ), jnp.int32),
            pltpu.SemaphoreType.DMA,
            pltpu.SemaphoreType.DMA,
        ],
        **_SC_PARAMS,
    )
    clist = fa(src_pad, dst_pad, asrc, adst)

    fb = pl.kernel(
        _phase_b_body_factory(D, LISTSZ),
        out_type=jax.ShapeDtypeStruct((NC, HALF, W), jnp.float32),
        mesh=mesh,
        scratch_types=[
            pltpu.VMEM((2, K), jnp.int32),
            pltpu.VMEM((K,), jnp.int32),
            pltpu.VMEM((K,), jnp.int32),
            pltpu.VMEM((K,), jnp.float32),
            pltpu.VMEM((K, D), jnp.float32),
            pltpu.SemaphoreType.DMA,
            pltpu.VMEM((2, K), jnp.int32),
            pltpu.VMEM((K,), jnp.int32),
            pltpu.VMEM((K,), jnp.int32),
            pltpu.VMEM((K,), jnp.float32),
            pltpu.VMEM((K, D), jnp.float32),
            pltpu.SemaphoreType.DMA,
            pltpu.VMEM((K, W), jnp.float32),
            pltpu.VMEM((K,), jnp.int32),
            pltpu.SemaphoreType.DMA,
            pltpu.VMEM_SHARED((ACC_R, W), jnp.float32),
        ],
        **_SC_PARAMS,
    )
    acc = fb(clist, h)
    return acc.reshape(NC * HALF, W)


def _mm_attn_body(x_ref, w_ref, asrc_ref, adst_ref, h_ref, as_ref, ad_ref):
    h = x_ref[...] @ w_ref[...]
    h_ref[...] = h
    as_ref[...] = jnp.sum(h * asrc_ref[...], axis=-1)
    ad_ref[...] = jnp.sum(h * adst_ref[...], axis=-1)


def _mm_attn(x_pad, W, att_src, att_dst):
    n, d_in = x_pad.shape
    d_out = W.shape[1]
    blk = 1024
    grid = n // blk
    return pl.pallas_call(
        _mm_attn_body,
        grid=(grid,),
        in_specs=[
            pl.BlockSpec((blk, d_in), lambda i: (i, 0)),
            pl.BlockSpec((d_in, d_out), lambda i: (0, 0)),
            pl.BlockSpec((1, d_out), lambda i: (0, 0)),
            pl.BlockSpec((1, d_out), lambda i: (0, 0)),
        ],
        out_specs=[
            pl.BlockSpec((blk, d_out), lambda i: (i, 0)),
            pl.BlockSpec((blk,), lambda i: (i,)),
            pl.BlockSpec((blk,), lambda i: (i,)),
        ],
        out_shape=[
            jax.ShapeDtypeStruct((n, d_out), jnp.float32),
            jax.ShapeDtypeStruct((n,), jnp.float32),
            jax.ShapeDtypeStruct((n,), jnp.float32),
        ],
    )(x_pad, W, att_src.reshape(1, -1), att_dst.reshape(1, -1))


def _norm_mm_attn_body(acc_ref, b_ref, w_ref, asrc_ref, adst_ref,
                       h_ref, as_ref, ad_ref, *, d_prev):
    x = acc_ref[...]
    num = x[:, :d_prev]
    den = x[:, d_prev:d_prev + 1]
    h_in = jax.nn.relu(num / (den + 1e-16) + b_ref[...])
    h = h_in @ w_ref[...]
    h_ref[...] = h
    as_ref[...] = jnp.sum(h * asrc_ref[...], axis=-1)
    ad_ref[...] = jnp.sum(h * adst_ref[...], axis=-1)


def _norm_mm_attn(acc_pad, b, W, att_src, att_dst):
    n, w_in = acc_pad.shape
    d_prev = b.shape[0]
    d_out = W.shape[1]
    blk = 1024
    return pl.pallas_call(
        functools.partial(_norm_mm_attn_body, d_prev=d_prev),
        grid=(n // blk,),
        in_specs=[
            pl.BlockSpec((blk, w_in), lambda i: (i, 0)),
            pl.BlockSpec((1, d_prev), lambda i: (0, 0)),
            pl.BlockSpec((d_prev, d_out), lambda i: (0, 0)),
            pl.BlockSpec((1, d_out), lambda i: (0, 0)),
            pl.BlockSpec((1, d_out), lambda i: (0, 0)),
        ],
        out_specs=[
            pl.BlockSpec((blk, d_out), lambda i: (i, 0)),
            pl.BlockSpec((blk,), lambda i: (i,)),
            pl.BlockSpec((blk,), lambda i: (i,)),
        ],
        out_shape=[
            jax.ShapeDtypeStruct((n, d_out), jnp.float32),
            jax.ShapeDtypeStruct((n,), jnp.float32),
            jax.ShapeDtypeStruct((n,), jnp.float32),
        ],
    )(acc_pad, b.reshape(1, -1), W,
      att_src.reshape(1, -1), att_dst.reshape(1, -1))


def _norm_uv_body(acc_ref, b_ref, wlu_ref, wlv_ref, u_ref, v_ref, *, d_prev):
    x = acc_ref[...]
    num = x[:, :d_prev]
    den = x[:, d_prev:d_prev + 1]
    o = num / (den + 1e-16) + b_ref[...]
    u_ref[...] = jnp.sum(o * wlu_ref[...], axis=-1)
    v_ref[...] = jnp.sum(o * wlv_ref[...], axis=-1)


def _norm_uv(acc_pad, b, Wl):
    n, w_in = acc_pad.shape
    d_prev = b.shape[0]
    blk = 1024
    return pl.pallas_call(
        functools.partial(_norm_uv_body, d_prev=d_prev),
        grid=(n // blk,),
        in_specs=[
            pl.BlockSpec((blk, w_in), lambda i: (i, 0)),
            pl.BlockSpec((1, d_prev), lambda i: (0, 0)),
            pl.BlockSpec((1, d_prev), lambda i: (0, 0)),
            pl.BlockSpec((1, d_prev), lambda i: (0, 0)),
        ],
        out_specs=[
            pl.BlockSpec((blk,), lambda i: (i,)),
            pl.BlockSpec((blk,), lambda i: (i,)),
        ],
        out_shape=[
            jax.ShapeDtypeStruct((n,), jnp.float32),
            jax.ShapeDtypeStruct((n,), jnp.float32),
        ],
    )(acc_pad, b.reshape(1, -1),
      Wl[:d_prev, 0].reshape(1, -1), Wl[d_prev:, 0].reshape(1, -1))


def kernel(g, features, mask, W1, att_src1, att_dst1, b1,
           W2, att_src2, att_dst2, b2, Wl, bl):
    n = features.shape[0]
    e = g.shape[1]
    e_pad = ((e + n + NS * BS - 1) // (NS * BS)) * (NS * BS)
    loop = jnp.arange(n, dtype=g.dtype)
    src_pad = jnp.concatenate(
        [g[0], loop, jnp.zeros((e_pad - e - n,), g.dtype)])
    dst_pad = jnp.concatenate(
        [g[1], loop, jnp.full((e_pad - e - n,), n, g.dtype)])

    x_pad = jnp.zeros((N_PAD, features.shape[1]), jnp.float32).at[:n].set(features)

    h1, as1, ad1 = _mm_attn(x_pad, W1, att_src1, att_dst1)
    acc1 = _edge_aggregate_sc(h1, as1[:NA], ad1[:NA], src_pad, dst_pad)
    acc1_pad = jnp.zeros((N_PAD, acc1.shape[1]), jnp.float32).at[:n].set(acc1)

    h2, as2, ad2 = _norm_mm_attn(acc1_pad, b1, W2, att_src2, att_dst2)
    acc2 = _edge_aggregate_sc(h2, as2[:NA], ad2[:NA], src_pad, dst_pad)
    acc2_pad = jnp.zeros((N_PAD, acc2.shape[1]), jnp.float32).at[:n].set(acc2)

    u, v = _norm_uv(acc2_pad, b2, Wl)
    u10 = u + bl[0]
    v10 = v
    nm = mask.shape[0]
    m_pad = ((nm + NW * 16 - 1) // (NW * 16)) * (NW * 16)
    m0 = jnp.pad(mask[:, 0], (0, m_pad - nm))
    m1 = jnp.pad(mask[:, 1], (0, m_pad - nm))
    p = _pair_score(m0, m1, u10, v10)
    return p[:nm, None]
